# baseline (speedup 1.0000x reference)
import functools

import jax
import jax.numpy as jnp
from jax import lax
from jax.experimental import pallas as pl
from jax.experimental.pallas import tpu as pltpu
from jax.experimental.pallas import tpu_sc as plsc

_TAU = 0.5
_DELTA = 3.0
_PEAK = 7
_NEIGH = 9
_TOPK = 200
_SLOTS = 256


def _shift_rows(x, d):
    h, w = x.shape
    if d == 0:
        return x
    z = jnp.zeros((abs(d), w), x.dtype)
    if d > 0:
        return jnp.concatenate([x[d:, :], z], axis=0)
    return jnp.concatenate([z, x[:d, :]], axis=0)


def _shift_cols(x, d):
    h, w = x.shape
    if d == 0:
        return x
    z = jnp.zeros((h, abs(d)), x.dtype)
    if d > 0:
        return jnp.concatenate([x[:, d:], z], axis=1)
    return jnp.concatenate([z, x[:, :d]], axis=1)


def _excl_prefix_rowmajor(x):
    h, w = x.shape
    c = x
    s = 1
    while s < w:
        c = c + _shift_cols(c, -s)
        s *= 2
    row_tot = c[:, w - 1:w]
    t = row_tot
    s = 1
    while s < h:
        t = t + _shift_rows(t, -s)
        s *= 2
    return (t - row_tot) + (c - x)


def _select_kernel(attn_ref, beta_ref, rank_ref, ax_ref, ay_ref):
    B, H, W = attn_ref.shape
    pr = _PEAK // 2
    nr = _NEIGH // 2
    beta = jnp.abs(beta_ref[0, 0]) + 1e-6

    m_all = jnp.maximum(attn_ref[...], 0.0)
    mmax = jnp.max(m_all)
    scale = jnp.where(mmax > 0, 1.0 / (mmax + 1e-8), 1.0)

    bits_l = []
    for b in range(B):
        m_th_raw = m_all[b] * scale
        m_th = jnp.where(m_th_raw > _TAU, m_th_raw, 0.0)

        rp = m_th
        for d in range(1, pr + 1):
            rp = jnp.maximum(
                rp, jnp.maximum(_shift_rows(m_th, d), _shift_rows(m_th, -d)))
        pooled = rp
        for d in range(1, pr + 1):
            pooled = jnp.maximum(
                pooled, jnp.maximum(_shift_cols(rp, d), _shift_cols(rp, -d)))

        is_peak = (m_th == pooled) & (m_th > 0)
        sc0 = jnp.where(is_peak, m_th, 0.0)
        bits_l.append(lax.bitcast_convert_type(sc0, jnp.int32))

        cs = m_th
        for d in range(1, nr + 1):
            cs = cs + _shift_rows(m_th, d) + _shift_rows(m_th, -d)
        ssx = jnp.zeros_like(m_th)
        for d in range(-nr, nr + 1):
            if d != 0:
                ssx = ssx + float(d * d) * _shift_cols(cs, d)
        sw = jnp.maximum(jnp.sqrt(jnp.maximum(ssx, 1e-8)), _DELTA)
        ax_ref[b] = 1.0 / (beta * sw * sw)

        rs = m_th
        for d in range(1, nr + 1):
            rs = rs + _shift_cols(m_th, d) + _shift_cols(m_th, -d)
        ssy = jnp.zeros_like(m_th)
        for d in range(-nr, nr + 1):
            if d != 0:
                ssy = ssy + float(d * d) * _shift_rows(rs, d)
        sh = jnp.maximum(jnp.sqrt(jnp.maximum(ssy, 1e-8)), _DELTA)
        ay_ref[b] = 1.0 / (beta * sh * sh)

    t_l = [jnp.int32(0) for _ in range(B)]
    for bit in range(30, -1, -1):
        for b in range(B):
            tc = t_l[b] | jnp.int32(1 << bit)
            cnt = jnp.sum((bits_l[b] >= tc).astype(jnp.int32))
            t_l[b] = jnp.where(cnt >= _TOPK, tc, t_l[b])

    for b in range(B):
        bits = bits_l[b]
        tt = t_l[b]
        gt = bits > tt
        n1 = jnp.sum(gt.astype(jnp.int32))
        m = _TOPK - n1
        tie = bits == tt
        tie_rank = _excl_prefix_rowmajor(tie.astype(jnp.int32))
        sel = gt | (tie & (tie_rank < m))
        rank = _excl_prefix_rowmajor(sel.astype(jnp.int32))
        rank_ref[b] = jnp.where(sel, rank, jnp.int32(H * W))


def _run_select(attn, beta, interpret=False):
    B, H, W = attn.shape
    return pl.pallas_call(
        _select_kernel,
        out_shape=(
            jax.ShapeDtypeStruct((B, H, W), jnp.int32),
            jax.ShapeDtypeStruct((B, H, W), jnp.float32),
            jax.ShapeDtypeStruct((B, H, W), jnp.float32),
        ),
        in_specs=[
            pl.BlockSpec((B, H, W), lambda: (0, 0, 0)),
            pl.BlockSpec((1, 1), lambda: (0, 0)),
        ],
        out_specs=(
            pl.BlockSpec((B, H, W), lambda: (0, 0, 0)),
            pl.BlockSpec((B, H, W), lambda: (0, 0, 0)),
            pl.BlockSpec((B, H, W), lambda: (0, 0, 0)),
        ),
        interpret=interpret,
    )(attn, beta.reshape(1, 1))


def _make_compact_kernel(B, HW):
    TPI = 8
    CHUNK = HW // TPI
    mesh = plsc.VectorSubcoreMesh(core_axis_name="c", subcore_axis_name="s")

    @functools.partial(
        pl.kernel,
        out_type=[jax.ShapeDtypeStruct((B * TPI * _SLOTS,), jnp.float32)
                  for _ in range(4)],
        mesh=mesh,
        compiler_params=pltpu.CompilerParams(needs_layout_passes=False),
        scratch_types=[
            pltpu.VMEM((CHUNK,), jnp.int32),
            pltpu.VMEM((CHUNK,), jnp.float32),
            pltpu.VMEM((CHUNK,), jnp.float32),
            pltpu.VMEM((_SLOTS,), jnp.float32),
            pltpu.VMEM((_SLOTS,), jnp.float32),
            pltpu.VMEM((_SLOTS,), jnp.float32),
            pltpu.VMEM((_SLOTS,), jnp.float32),
        ],
    )
    def compact(rank_hbm, axm_hbm, aym_hbm, cxo, cyo, axo, ayo,
                rankv, axv, ayv, cxb, cyb, axb, ayb):
        nc = plsc.get_sparse_core_info().num_cores
        wid = lax.axis_index("s") * nc + lax.axis_index("c")
        base = wid * CHUNK

        neg1 = jnp.full((16,), -1.0, jnp.float32)

        def init_body(j, carry):
            cxb[pl.ds(j * 16, 16)] = neg1
            cyb[pl.ds(j * 16, 16)] = neg1
            axb[pl.ds(j * 16, 16)] = neg1
            ayb[pl.ds(j * 16, 16)] = neg1
            return carry

        lax.fori_loop(0, _SLOTS // 16, init_body, 0)

        pltpu.sync_copy(rank_hbm.at[pl.ds(base, CHUNK)], rankv)
        pltpu.sync_copy(axm_hbm.at[pl.ds(base, CHUNK)], axv)
        pltpu.sync_copy(aym_hbm.at[pl.ds(base, CHUNK)], ayv)
        pos0 = base - (wid // TPI) * HW

        def vec_body(i, carry):
            rv = rankv[pl.ds(i * 16, 16)]
            mask = rv < _SLOTS
            idx = jnp.minimum(rv, _SLOTS - 1)
            pos = lax.iota(jnp.int32, 16) + (pos0 + i * 16)
            cxv = (pos & 127).astype(jnp.float32)
            cyv = (pos >> 7).astype(jnp.float32)
            plsc.store_scatter(cxb, [idx], cxv, mask=mask)
            plsc.store_scatter(cyb, [idx], cyv, mask=mask)
            plsc.store_scatter(axb, [idx], axv[pl.ds(i * 16, 16)], mask=mask)
            plsc.store_scatter(ayb, [idx], ayv[pl.ds(i * 16, 16)], mask=mask)
            return carry

        lax.fori_loop(0, CHUNK // 16, vec_body, 0)
        out_off = wid * _SLOTS
        pltpu.sync_copy(cxb, cxo.at[pl.ds(out_off, _SLOTS)])
        pltpu.sync_copy(cyb, cyo.at[pl.ds(out_off, _SLOTS)])
        pltpu.sync_copy(axb, axo.at[pl.ds(out_off, _SLOTS)])
        pltpu.sync_copy(ayb, ayo.at[pl.ds(out_off, _SLOTS)])

    return compact


def _gaussmod_kernel(cx_ref, cy_ref, ax_ref, ay_ref, agg_ref, gamma_ref,
                     out_ref, g_s, cxm, cym, axm, aym):
    B, TPI, S, _ = cx_ref.shape
    H, W = out_ref.shape[2], out_ref.shape[3]
    c = pl.program_id(1)

    @pl.when(c == 0)
    def _():
        lane_f = jax.lax.broadcasted_iota(
            jnp.int32, (1, W), 1).astype(jnp.float32)
        col_f = jax.lax.broadcasted_iota(
            jnp.int32, (H, 1), 0).astype(jnp.float32)

        for ref, mref in ((cx_ref, cxm), (cy_ref, cym),
                          (ax_ref, axm), (ay_ref, aym)):
            acc = ref[0, 0]
            for t in range(1, TPI):
                acc = jnp.maximum(acc, ref[0, t])
            mref[...] = acc

        UNROLL = 4

        def body(k2, glog):
            contrib = None
            for u in range(UNROLL):
                k = k2 * UNROLL + u
                cxk = cxm[pl.ds(k, 1), :]
                cyk = cym[pl.ds(k, 1), :]
                axk = axm[pl.ds(k, 1), :]
                ayk = aym[pl.ds(k, 1), :]
                dx = lane_f - cxk
                dy = col_f - cyk
                term = (-(dy * dy) * ayk) + (-(dx * dx) * axk)
                contrib = term if contrib is None else jnp.maximum(
                    contrib, term)
            return jnp.maximum(glog, contrib)

        glog = jax.lax.fori_loop(
            0, _TOPK // UNROLL, body,
            jnp.full((H, W), -jnp.inf, jnp.float32))
        g_s[...] = jnp.exp(glog)

    g = g_s[...][None, None, :, :]
    a = agg_ref[...]
    out_ref[...] = a + gamma_ref[0, 0] * (g * a)


def _run_gaussmod(cx, cy, ax, ay, agg, gamma, interpret=False):
    B, C, H, W = agg.shape
    TPI = 8
    CB = 64
    pspec = pl.BlockSpec((1, TPI, _SLOTS, 1), lambda b, c: (b, 0, 0, 0))
    return pl.pallas_call(
        _gaussmod_kernel,
        grid=(B, C // CB),
        out_shape=jax.ShapeDtypeStruct((B, C, H, W), jnp.float32),
        in_specs=[pspec] * 4 + [
            pl.BlockSpec((1, CB, H, W), lambda b, c: (b, c, 0, 0)),
            pl.BlockSpec((1, 1), lambda b, c: (0, 0)),
        ],
        out_specs=pl.BlockSpec((1, CB, H, W), lambda b, c: (b, c, 0, 0)),
        scratch_shapes=[pltpu.VMEM((H, W), jnp.float32)] + [
            pltpu.VMEM((_SLOTS, 1), jnp.float32) for _ in range(4)],
        interpret=interpret,
    )(cx.reshape(B, TPI, _SLOTS, 1), cy.reshape(B, TPI, _SLOTS, 1),
      ax.reshape(B, TPI, _SLOTS, 1), ay.reshape(B, TPI, _SLOTS, 1),
      agg, gamma.reshape(1, 1))


@jax.jit
def kernel(agg_detection_feats, detection_attn_map, beta, gamma):
    B, C, H, W = agg_detection_feats.shape
    beta = jnp.asarray(beta, jnp.float32)
    gamma = jnp.asarray(gamma, jnp.float32)

    rank, axm, aym = _run_select(detection_attn_map, beta)
    compact = _make_compact_kernel(B, H * W)
    cx, cy, ax, ay = compact(
        rank.reshape(B * H * W), axm.reshape(B * H * W),
        aym.reshape(B * H * W))
    return _run_gaussmod(cx, cy, ax, ay, agg_detection_feats, gamma)


# speedup vs baseline: 1.0120x; 1.0120x over previous
import functools

import jax
import jax.numpy as jnp
from jax import lax
from jax.experimental import pallas as pl
from jax.experimental.pallas import tpu as pltpu
from jax.experimental.pallas import tpu_sc as plsc

_TAU = 0.5
_DELTA = 3.0
_PEAK = 7
_NEIGH = 9
_TOPK = 200
_SLOTS = 256


def _shift_rows(x, d):
    h, w = x.shape
    if d == 0:
        return x
    z = jnp.zeros((abs(d), w), x.dtype)
    if d > 0:
        return jnp.concatenate([x[d:, :], z], axis=0)
    return jnp.concatenate([z, x[:d, :]], axis=0)


def _shift_cols(x, d):
    h, w = x.shape
    if d == 0:
        return x
    z = jnp.zeros((h, abs(d)), x.dtype)
    if d > 0:
        return jnp.concatenate([x[:, d:], z], axis=1)
    return jnp.concatenate([z, x[:, :d]], axis=1)


def _excl_prefix_rowmajor(x):
    h, w = x.shape
    c = x
    s = 1
    while s < w:
        c = c + _shift_cols(c, -s)
        s *= 2
    row_tot = c[:, w - 1:w]
    t = row_tot
    s = 1
    while s < h:
        t = t + _shift_rows(t, -s)
        s *= 2
    return (t - row_tot) + (c - x)


def _select_kernel(attn_ref, beta_ref, rank_ref, ax_ref, ay_ref):
    B, H, W = attn_ref.shape
    pr = _PEAK // 2
    nr = _NEIGH // 2
    beta = jnp.abs(beta_ref[0, 0]) + 1e-6

    m_all = jnp.maximum(attn_ref[...], 0.0)
    mmax = jnp.max(m_all)
    scale = jnp.where(mmax > 0, 1.0 / (mmax + 1e-8), 1.0)

    bits_l = []
    for b in range(B):
        m_th_raw = m_all[b] * scale
        m_th = jnp.where(m_th_raw > _TAU, m_th_raw, 0.0)

        rp = m_th
        for d in range(1, pr + 1):
            rp = jnp.maximum(
                rp, jnp.maximum(_shift_rows(m_th, d), _shift_rows(m_th, -d)))
        pooled = rp
        for d in range(1, pr + 1):
            pooled = jnp.maximum(
                pooled, jnp.maximum(_shift_cols(rp, d), _shift_cols(rp, -d)))

        is_peak = (m_th == pooled) & (m_th > 0)
        sc0 = jnp.where(is_peak, m_th, 0.0)
        bits_l.append(lax.bitcast_convert_type(sc0, jnp.int32))

        cs = m_th
        for d in range(1, nr + 1):
            cs = cs + _shift_rows(m_th, d) + _shift_rows(m_th, -d)
        ssx = jnp.zeros_like(m_th)
        for d in range(-nr, nr + 1):
            if d != 0:
                ssx = ssx + float(d * d) * _shift_cols(cs, d)
        sw = jnp.maximum(jnp.sqrt(jnp.maximum(ssx, 1e-8)), _DELTA)
        ax_ref[b] = 1.0 / (beta * sw * sw)

        rs = m_th
        for d in range(1, nr + 1):
            rs = rs + _shift_cols(m_th, d) + _shift_cols(m_th, -d)
        ssy = jnp.zeros_like(m_th)
        for d in range(-nr, nr + 1):
            if d != 0:
                ssy = ssy + float(d * d) * _shift_rows(rs, d)
        sh = jnp.maximum(jnp.sqrt(jnp.maximum(ssy, 1e-8)), _DELTA)
        ay_ref[b] = 1.0 / (beta * sh * sh)

    t_l = [jnp.int32(0) for _ in range(B)]
    for bit in range(30, -1, -1):
        for b in range(B):
            tc = t_l[b] | jnp.int32(1 << bit)
            cnt = jnp.sum((bits_l[b] >= tc).astype(jnp.int32))
            t_l[b] = jnp.where(cnt >= _TOPK, tc, t_l[b])

    for b in range(B):
        bits = bits_l[b]
        tt = t_l[b]
        gt = bits > tt
        n1 = jnp.sum(gt.astype(jnp.int32))
        m = _TOPK - n1
        tie = bits == tt
        tie_rank = _excl_prefix_rowmajor(tie.astype(jnp.int32))
        sel = gt | (tie & (tie_rank < m))
        rank = _excl_prefix_rowmajor(sel.astype(jnp.int32))
        rank_ref[b] = jnp.where(sel, rank, jnp.int32(H * W))


def _run_select(attn, beta, interpret=False):
    B, H, W = attn.shape
    return pl.pallas_call(
        _select_kernel,
        out_shape=(
            jax.ShapeDtypeStruct((B, H, W), jnp.int32),
            jax.ShapeDtypeStruct((B, H, W), jnp.float32),
            jax.ShapeDtypeStruct((B, H, W), jnp.float32),
        ),
        in_specs=[
            pl.BlockSpec((B, H, W), lambda: (0, 0, 0)),
            pl.BlockSpec((1, 1), lambda: (0, 0)),
        ],
        out_specs=(
            pl.BlockSpec((B, H, W), lambda: (0, 0, 0)),
            pl.BlockSpec((B, H, W), lambda: (0, 0, 0)),
            pl.BlockSpec((B, H, W), lambda: (0, 0, 0)),
        ),
        interpret=interpret,
    )(attn, beta.reshape(1, 1))


def _make_compact_kernel(B, HW):
    TPI = 8
    CHUNK = HW // TPI
    mesh = plsc.VectorSubcoreMesh(core_axis_name="c", subcore_axis_name="s")

    @functools.partial(
        pl.kernel,
        out_type=[jax.ShapeDtypeStruct((B * TPI * _SLOTS,), jnp.float32)
                  for _ in range(4)],
        mesh=mesh,
        compiler_params=pltpu.CompilerParams(needs_layout_passes=False),
        scratch_types=[
            pltpu.VMEM((CHUNK,), jnp.int32),
            pltpu.VMEM((CHUNK,), jnp.float32),
            pltpu.VMEM((CHUNK,), jnp.float32),
            pltpu.VMEM((_SLOTS,), jnp.float32),
            pltpu.VMEM((_SLOTS,), jnp.float32),
            pltpu.VMEM((_SLOTS,), jnp.float32),
            pltpu.VMEM((_SLOTS,), jnp.float32),
        ],
    )
    def compact(rank_hbm, axm_hbm, aym_hbm, cxo, cyo, axo, ayo,
                rankv, axv, ayv, cxb, cyb, axb, ayb):
        nc = plsc.get_sparse_core_info().num_cores
        wid = lax.axis_index("s") * nc + lax.axis_index("c")
        base = wid * CHUNK

        neg1 = jnp.full((16,), -1.0, jnp.float32)

        def init_body(j, carry):
            cxb[pl.ds(j * 16, 16)] = neg1
            cyb[pl.ds(j * 16, 16)] = neg1
            axb[pl.ds(j * 16, 16)] = neg1
            ayb[pl.ds(j * 16, 16)] = neg1
            return carry

        lax.fori_loop(0, _SLOTS // 16, init_body, 0)

        pltpu.sync_copy(rank_hbm.at[pl.ds(base, CHUNK)], rankv)
        pltpu.sync_copy(axm_hbm.at[pl.ds(base, CHUNK)], axv)
        pltpu.sync_copy(aym_hbm.at[pl.ds(base, CHUNK)], ayv)
        pos0 = base - (wid // TPI) * HW

        def vec_body(i, carry):
            rv = rankv[pl.ds(i * 16, 16)]
            mask = rv < _SLOTS
            idx = jnp.minimum(rv, _SLOTS - 1)
            pos = lax.iota(jnp.int32, 16) + (pos0 + i * 16)
            cxv = (pos & 127).astype(jnp.float32)
            cyv = (pos >> 7).astype(jnp.float32)
            plsc.store_scatter(cxb, [idx], cxv, mask=mask)
            plsc.store_scatter(cyb, [idx], cyv, mask=mask)
            plsc.store_scatter(axb, [idx], axv[pl.ds(i * 16, 16)], mask=mask)
            plsc.store_scatter(ayb, [idx], ayv[pl.ds(i * 16, 16)], mask=mask)
            return carry

        lax.fori_loop(0, CHUNK // 16, vec_body, 0)
        out_off = wid * _SLOTS
        pltpu.sync_copy(cxb, cxo.at[pl.ds(out_off, _SLOTS)])
        pltpu.sync_copy(cyb, cyo.at[pl.ds(out_off, _SLOTS)])
        pltpu.sync_copy(axb, axo.at[pl.ds(out_off, _SLOTS)])
        pltpu.sync_copy(ayb, ayo.at[pl.ds(out_off, _SLOTS)])

    return compact


def _gaussmod_kernel(cx_ref, cy_ref, ax_ref, ay_ref, agg_ref, gamma_ref,
                     out_ref, g_s, cxm, cym, axm, aym):
    B, TPI, S, _ = cx_ref.shape
    H, W = out_ref.shape[2], out_ref.shape[3]
    b = pl.program_id(0)
    c = pl.program_id(1)
    step = b * pl.num_programs(1) + c

    @pl.when(step == 0)
    def _():
        lane_f = jax.lax.broadcasted_iota(
            jnp.int32, (1, W), 1).astype(jnp.float32)
        col_f = jax.lax.broadcasted_iota(
            jnp.int32, (H, 1), 0).astype(jnp.float32)

        for bb in range(B):
            for ref, mref in ((cx_ref, cxm), (cy_ref, cym),
                              (ax_ref, axm), (ay_ref, aym)):
                acc = ref[bb, 0]
                for t in range(1, TPI):
                    acc = jnp.maximum(acc, ref[bb, t])
                mref[...] = acc

            UNROLL = 4

            def body(k2, glog):
                contrib = None
                for u in range(UNROLL):
                    k = k2 * UNROLL + u
                    cxk = cxm[pl.ds(k, 1), :]
                    cyk = cym[pl.ds(k, 1), :]
                    axk = axm[pl.ds(k, 1), :]
                    ayk = aym[pl.ds(k, 1), :]
                    dx = lane_f - cxk
                    dy = col_f - cyk
                    term = (-(dy * dy) * ayk) + (-(dx * dx) * axk)
                    contrib = term if contrib is None else jnp.maximum(
                        contrib, term)
                return jnp.maximum(glog, contrib)

            glog = jax.lax.fori_loop(
                0, _TOPK // UNROLL, body,
                jnp.full((H, W), -jnp.inf, jnp.float32))
            g_s[bb] = jnp.exp(glog)

    g = g_s[b][None, None, :, :]
    a = agg_ref[...]
    out_ref[...] = a + gamma_ref[0, 0] * (g * a)


def _run_gaussmod(cx, cy, ax, ay, agg, gamma, interpret=False):
    B, C, H, W = agg.shape
    TPI = 8
    CB = 64
    pspec = pl.BlockSpec((B, TPI, _SLOTS, 1), lambda b, c: (0, 0, 0, 0))
    return pl.pallas_call(
        _gaussmod_kernel,
        grid=(B, C // CB),
        out_shape=jax.ShapeDtypeStruct((B, C, H, W), jnp.float32),
        in_specs=[pspec] * 4 + [
            pl.BlockSpec((1, CB, H, W), lambda b, c: (b, c, 0, 0)),
            pl.BlockSpec((1, 1), lambda b, c: (0, 0)),
        ],
        out_specs=pl.BlockSpec((1, CB, H, W), lambda b, c: (b, c, 0, 0)),
        scratch_shapes=[pltpu.VMEM((B, H, W), jnp.float32)] + [
            pltpu.VMEM((_SLOTS, 1), jnp.float32) for _ in range(4)],
        interpret=interpret,
    )(cx.reshape(B, TPI, _SLOTS, 1), cy.reshape(B, TPI, _SLOTS, 1),
      ax.reshape(B, TPI, _SLOTS, 1), ay.reshape(B, TPI, _SLOTS, 1),
      agg, gamma.reshape(1, 1))


@jax.jit
def kernel(agg_detection_feats, detection_attn_map, beta, gamma):
    B, C, H, W = agg_detection_feats.shape
    beta = jnp.asarray(beta, jnp.float32)
    gamma = jnp.asarray(gamma, jnp.float32)

    rank, axm, aym = _run_select(detection_attn_map, beta)
    compact = _make_compact_kernel(B, H * W)
    cx, cy, ax, ay = compact(
        rank.reshape(B * H * W), axm.reshape(B * H * W),
        aym.reshape(B * H * W))
    return _run_gaussmod(cx, cy, ax, ay, agg_detection_feats, gamma)


# speedup vs baseline: 1.0920x; 1.0791x over previous
import functools

import jax
import jax.numpy as jnp
from jax import lax
from jax.experimental import pallas as pl
from jax.experimental.pallas import tpu as pltpu
from jax.experimental.pallas import tpu_sc as plsc

_TAU = 0.5
_DELTA = 3.0
_PEAK = 7
_NEIGH = 9
_TOPK = 200
_SLOTS = 256


def _shift_rows(x, d):
    h, w = x.shape
    if d == 0:
        return x
    z = jnp.zeros((abs(d), w), x.dtype)
    if d > 0:
        return jnp.concatenate([x[d:, :], z], axis=0)
    return jnp.concatenate([z, x[:d, :]], axis=0)


def _shift_cols(x, d):
    h, w = x.shape
    if d == 0:
        return x
    z = jnp.zeros((h, abs(d)), x.dtype)
    if d > 0:
        return jnp.concatenate([x[:, d:], z], axis=1)
    return jnp.concatenate([z, x[:, :d]], axis=1)


def _excl_prefix_rowmajor(x):
    h, w = x.shape
    c = x
    s = 1
    while s < w:
        c = c + _shift_cols(c, -s)
        s *= 2
    row_tot = c[:, w - 1:w]
    t = row_tot
    s = 1
    while s < h:
        t = t + _shift_rows(t, -s)
        s *= 2
    return (t - row_tot) + (c - x)


def _select_kernel(attn_ref, beta_ref, rank_ref, ax_ref, ay_ref):
    B, H, W = attn_ref.shape
    pr = _PEAK // 2
    nr = _NEIGH // 2
    beta = jnp.abs(beta_ref[0, 0]) + 1e-6

    m_all = jnp.maximum(attn_ref[...], 0.0)
    mmax = jnp.max(m_all)
    scale = jnp.where(mmax > 0, 1.0 / (mmax + 1e-8), 1.0)

    bits_l = []
    for b in range(B):
        m_th_raw = m_all[b] * scale
        m_th = jnp.where(m_th_raw > _TAU, m_th_raw, 0.0)

        rp = m_th
        for d in range(1, pr + 1):
            rp = jnp.maximum(
                rp, jnp.maximum(_shift_rows(m_th, d), _shift_rows(m_th, -d)))
        pooled = rp
        for d in range(1, pr + 1):
            pooled = jnp.maximum(
                pooled, jnp.maximum(_shift_cols(rp, d), _shift_cols(rp, -d)))

        is_peak = (m_th == pooled) & (m_th > 0)
        sc0 = jnp.where(is_peak, m_th, 0.0)
        bits_l.append(lax.bitcast_convert_type(sc0, jnp.int32))

        cs = m_th
        for d in range(1, nr + 1):
            cs = cs + _shift_rows(m_th, d) + _shift_rows(m_th, -d)
        ssx = jnp.zeros_like(m_th)
        for d in range(-nr, nr + 1):
            if d != 0:
                ssx = ssx + float(d * d) * _shift_cols(cs, d)
        sw = jnp.maximum(jnp.sqrt(jnp.maximum(ssx, 1e-8)), _DELTA)
        ax_ref[b] = 1.0 / (beta * sw * sw)

        rs = m_th
        for d in range(1, nr + 1):
            rs = rs + _shift_cols(m_th, d) + _shift_cols(m_th, -d)
        ssy = jnp.zeros_like(m_th)
        for d in range(-nr, nr + 1):
            if d != 0:
                ssy = ssy + float(d * d) * _shift_rows(rs, d)
        sh = jnp.maximum(jnp.sqrt(jnp.maximum(ssy, 1e-8)), _DELTA)
        ay_ref[b] = 1.0 / (beta * sh * sh)

    t_l = []
    for b in range(B):
        cnt0 = jnp.sum((bits_l[b] >= jnp.int32(0x3F000000)).astype(jnp.int32))
        t_l.append(jnp.where(cnt0 >= _TOPK, jnp.int32(0x3F000000),
                             jnp.int32(0)))
    for bit in range(23, -1, -1):
        for b in range(B):
            tc = t_l[b] | jnp.int32(1 << bit)
            cnt = jnp.sum((bits_l[b] >= tc).astype(jnp.int32))
            t_l[b] = jnp.where(cnt >= _TOPK, tc, t_l[b])

    for b in range(B):
        bits = bits_l[b]
        tt = t_l[b]
        gt = bits > tt
        n1 = jnp.sum(gt.astype(jnp.int32))
        m = _TOPK - n1
        tie = bits == tt
        tie_rank = _excl_prefix_rowmajor(tie.astype(jnp.int32))
        sel = gt | (tie & (tie_rank < m))
        rank = _excl_prefix_rowmajor(sel.astype(jnp.int32))
        rank_ref[b] = jnp.where(sel, rank, jnp.int32(H * W))


def _run_select(attn, beta, interpret=False):
    B, H, W = attn.shape
    return pl.pallas_call(
        _select_kernel,
        out_shape=(
            jax.ShapeDtypeStruct((B, H, W), jnp.int32),
            jax.ShapeDtypeStruct((B, H, W), jnp.float32),
            jax.ShapeDtypeStruct((B, H, W), jnp.float32),
        ),
        in_specs=[
            pl.BlockSpec((B, H, W), lambda: (0, 0, 0)),
            pl.BlockSpec((1, 1), lambda: (0, 0)),
        ],
        out_specs=(
            pl.BlockSpec((B, H, W), lambda: (0, 0, 0)),
            pl.BlockSpec((B, H, W), lambda: (0, 0, 0)),
            pl.BlockSpec((B, H, W), lambda: (0, 0, 0)),
        ),
        interpret=interpret,
    )(attn, beta.reshape(1, 1))


def _make_compact_kernel(B, HW):
    TPI = 8
    CHUNK = HW // TPI
    mesh = plsc.VectorSubcoreMesh(core_axis_name="c", subcore_axis_name="s")

    @functools.partial(
        pl.kernel,
        out_type=[jax.ShapeDtypeStruct((B * TPI * _SLOTS,), jnp.float32)
                  for _ in range(4)],
        mesh=mesh,
        compiler_params=pltpu.CompilerParams(needs_layout_passes=False),
        scratch_types=[
            pltpu.VMEM((CHUNK,), jnp.int32),
            pltpu.VMEM((CHUNK,), jnp.float32),
            pltpu.VMEM((CHUNK,), jnp.float32),
            pltpu.VMEM((_SLOTS,), jnp.float32),
            pltpu.VMEM((_SLOTS,), jnp.float32),
            pltpu.VMEM((_SLOTS,), jnp.float32),
            pltpu.VMEM((_SLOTS,), jnp.float32),
        ],
    )
    def compact(rank_hbm, axm_hbm, aym_hbm, cxo, cyo, axo, ayo,
                rankv, axv, ayv, cxb, cyb, axb, ayb):
        nc = plsc.get_sparse_core_info().num_cores
        wid = lax.axis_index("s") * nc + lax.axis_index("c")
        base = wid * CHUNK

        neg1 = jnp.full((16,), -1.0, jnp.float32)

        def init_body(j, carry):
            cxb[pl.ds(j * 16, 16)] = neg1
            cyb[pl.ds(j * 16, 16)] = neg1
            axb[pl.ds(j * 16, 16)] = neg1
            ayb[pl.ds(j * 16, 16)] = neg1
            return carry

        lax.fori_loop(0, _SLOTS // 16, init_body, 0)

        pltpu.sync_copy(rank_hbm.at[pl.ds(base, CHUNK)], rankv)
        pltpu.sync_copy(axm_hbm.at[pl.ds(base, CHUNK)], axv)
        pltpu.sync_copy(aym_hbm.at[pl.ds(base, CHUNK)], ayv)
        pos0 = base - (wid // TPI) * HW

        def vec_body(i, carry):
            rv = rankv[pl.ds(i * 16, 16)]
            mask = rv < _SLOTS
            idx = jnp.minimum(rv, _SLOTS - 1)
            pos = lax.iota(jnp.int32, 16) + (pos0 + i * 16)
            cxv = (pos & 127).astype(jnp.float32)
            cyv = (pos >> 7).astype(jnp.float32)
            plsc.store_scatter(cxb, [idx], cxv, mask=mask)
            plsc.store_scatter(cyb, [idx], cyv, mask=mask)
            plsc.store_scatter(axb, [idx], axv[pl.ds(i * 16, 16)], mask=mask)
            plsc.store_scatter(ayb, [idx], ayv[pl.ds(i * 16, 16)], mask=mask)
            return carry

        lax.fori_loop(0, CHUNK // 16, vec_body, 0)
        out_off = wid * _SLOTS
        pltpu.sync_copy(cxb, cxo.at[pl.ds(out_off, _SLOTS)])
        pltpu.sync_copy(cyb, cyo.at[pl.ds(out_off, _SLOTS)])
        pltpu.sync_copy(axb, axo.at[pl.ds(out_off, _SLOTS)])
        pltpu.sync_copy(ayb, ayo.at[pl.ds(out_off, _SLOTS)])

    return compact


def _gauss_kernel(cx_ref, cy_ref, ax_ref, ay_ref, g_ref,
                  cxm, cym, axm, aym, glog_s):
    B, TPI, S, _ = cx_ref.shape
    H, W = g_ref.shape[1], g_ref.shape[2]
    lane_f = jax.lax.broadcasted_iota(jnp.int32, (1, W), 1).astype(jnp.float32)
    col_f = jax.lax.broadcasted_iota(jnp.int32, (H, 1), 0).astype(jnp.float32)

    for b in range(B):
        for ref, mref in ((cx_ref, cxm), (cy_ref, cym),
                          (ax_ref, axm), (ay_ref, aym)):
            acc = ref[b, 0]
            for t in range(1, TPI):
                acc = jnp.maximum(acc, ref[b, t])
            mref[b] = acc

    UNROLL = 4
    for b in range(B):
        glog_s[b] = jnp.full((H, W), -jnp.inf, jnp.float32)

    def body(k2, carry):
        for b in range(B):
            contrib = None
            for u in range(UNROLL):
                k = k2 * UNROLL + u
                cxk = cxm[b, pl.ds(k, 1), :]
                cyk = cym[b, pl.ds(k, 1), :]
                axk = axm[b, pl.ds(k, 1), :]
                ayk = aym[b, pl.ds(k, 1), :]
                dx = lane_f - cxk
                dy = col_f - cyk
                term = (-(dy * dy) * ayk) + (-(dx * dx) * axk)
                contrib = term if contrib is None else jnp.maximum(
                    contrib, term)
            glog_s[b] = jnp.maximum(glog_s[b], contrib)
        return carry

    jax.lax.fori_loop(0, _TOPK // UNROLL, body, 0)
    for b in range(B):
        g_ref[b] = jnp.exp(glog_s[b])


def _run_gauss(cx, cy, ax, ay, B, H, W, interpret=False):
    TPI = 8
    spec = pl.BlockSpec((B, TPI, _SLOTS, 1), lambda: (0, 0, 0, 0))
    scratch = [pltpu.VMEM((B, _SLOTS, 1), jnp.float32) for _ in range(4)]
    scratch.append(pltpu.VMEM((B, H, W), jnp.float32))
    return pl.pallas_call(
        _gauss_kernel,
        out_shape=jax.ShapeDtypeStruct((B, H, W), jnp.float32),
        in_specs=[spec] * 4,
        out_specs=pl.BlockSpec((B, H, W), lambda: (0, 0, 0)),
        scratch_shapes=scratch,
        interpret=interpret,
    )(cx.reshape(B, TPI, _SLOTS, 1), cy.reshape(B, TPI, _SLOTS, 1),
      ax.reshape(B, TPI, _SLOTS, 1), ay.reshape(B, TPI, _SLOTS, 1))


def _mod_kernel(agg_ref, g_ref, gamma_ref, out_ref):
    g = g_ref[0][None, None, :, :]
    a = agg_ref[...]
    out_ref[...] = a + gamma_ref[0, 0] * (g * a)


def _run_mod(agg, g, gamma, interpret=False):
    B, C, H, W = agg.shape
    CB = 64
    return pl.pallas_call(
        _mod_kernel,
        grid=(B, C // CB),
        out_shape=jax.ShapeDtypeStruct((B, C, H, W), jnp.float32),
        in_specs=[
            pl.BlockSpec((1, CB, H, W), lambda b, c: (b, c, 0, 0)),
            pl.BlockSpec((1, H, W), lambda b, c: (b, 0, 0)),
            pl.BlockSpec((1, 1), lambda b, c: (0, 0)),
        ],
        out_specs=pl.BlockSpec((1, CB, H, W), lambda b, c: (b, c, 0, 0)),
        interpret=interpret,
    )(agg, g, gamma.reshape(1, 1))


@jax.jit
def kernel(agg_detection_feats, detection_attn_map, beta, gamma):
    B, C, H, W = agg_detection_feats.shape
    beta = jnp.asarray(beta, jnp.float32)
    gamma = jnp.asarray(gamma, jnp.float32)

    rank, axm, aym = _run_select(detection_attn_map, beta)
    compact = _make_compact_kernel(B, H * W)
    cx, cy, ax, ay = compact(
        rank.reshape(B * H * W), axm.reshape(B * H * W),
        aym.reshape(B * H * W))
    g = _run_gauss(cx, cy, ax, ay, B, H, W)
    return _run_mod(agg_detection_feats, g, gamma)


# speedup vs baseline: 1.1010x; 1.0082x over previous
import functools

import jax
import jax.numpy as jnp
from jax import lax
from jax.experimental import pallas as pl
from jax.experimental.pallas import tpu as pltpu
from jax.experimental.pallas import tpu_sc as plsc

_TAU = 0.5
_DELTA = 3.0
_PEAK = 7
_NEIGH = 9
_TOPK = 200
_SLOTS = 256


def _shift_rows(x, d):
    h, w = x.shape
    if d == 0:
        return x
    z = jnp.zeros((abs(d), w), x.dtype)
    if d > 0:
        return jnp.concatenate([x[d:, :], z], axis=0)
    return jnp.concatenate([z, x[:d, :]], axis=0)


def _shift_cols(x, d):
    h, w = x.shape
    if d == 0:
        return x
    z = jnp.zeros((h, abs(d)), x.dtype)
    if d > 0:
        return jnp.concatenate([x[:, d:], z], axis=1)
    return jnp.concatenate([z, x[:, :d]], axis=1)


def _excl_prefix_rowmajor(x):
    h, w = x.shape
    c = x
    s = 1
    while s < w:
        c = c + _shift_cols(c, -s)
        s *= 2
    row_tot = c[:, w - 1:w]
    t = row_tot
    s = 1
    while s < h:
        t = t + _shift_rows(t, -s)
        s *= 2
    return (t - row_tot) + (c - x)


def _select_kernel(attn_ref, beta_ref, rank_ref, ax_ref, ay_ref):
    B, H, W = attn_ref.shape
    pr = _PEAK // 2
    nr = _NEIGH // 2
    beta = jnp.abs(beta_ref[0, 0]) + 1e-6

    m_all = jnp.maximum(attn_ref[...], 0.0)
    mmax = jnp.max(m_all)
    scale = jnp.where(mmax > 0, 1.0 / (mmax + 1e-8), 1.0)

    bits_l = []
    for b in range(B):
        m_th_raw = m_all[b] * scale
        m_th = jnp.where(m_th_raw > _TAU, m_th_raw, 0.0)

        rp = m_th
        for d in range(1, pr + 1):
            rp = jnp.maximum(
                rp, jnp.maximum(_shift_rows(m_th, d), _shift_rows(m_th, -d)))
        pooled = rp
        for d in range(1, pr + 1):
            pooled = jnp.maximum(
                pooled, jnp.maximum(_shift_cols(rp, d), _shift_cols(rp, -d)))

        is_peak = (m_th == pooled) & (m_th > 0)
        sc0 = jnp.where(is_peak, m_th, 0.0)
        bits_l.append(lax.bitcast_convert_type(sc0, jnp.int32))

        cs = m_th
        for d in range(1, nr + 1):
            cs = cs + _shift_rows(m_th, d) + _shift_rows(m_th, -d)
        ssx = jnp.zeros_like(m_th)
        for d in range(-nr, nr + 1):
            if d != 0:
                ssx = ssx + float(d * d) * _shift_cols(cs, d)
        sw = jnp.maximum(jnp.sqrt(jnp.maximum(ssx, 1e-8)), _DELTA)
        ax_ref[b] = 1.0 / (beta * sw * sw)

        rs = m_th
        for d in range(1, nr + 1):
            rs = rs + _shift_cols(m_th, d) + _shift_cols(m_th, -d)
        ssy = jnp.zeros_like(m_th)
        for d in range(-nr, nr + 1):
            if d != 0:
                ssy = ssy + float(d * d) * _shift_rows(rs, d)
        sh = jnp.maximum(jnp.sqrt(jnp.maximum(ssy, 1e-8)), _DELTA)
        ay_ref[b] = 1.0 / (beta * sh * sh)

    t_l = []
    for b in range(B):
        cnt0 = jnp.sum((bits_l[b] >= jnp.int32(0x3F000000)).astype(jnp.int32))
        t_l.append(jnp.where(cnt0 >= _TOPK, jnp.int32(0x3F000000),
                             jnp.int32(0)))
    for bit in range(23, -1, -1):
        for b in range(B):
            tc = t_l[b] | jnp.int32(1 << bit)
            cnt = jnp.sum((bits_l[b] >= tc).astype(jnp.int32))
            t_l[b] = jnp.where(cnt >= _TOPK, tc, t_l[b])

    for b in range(B):
        bits = bits_l[b]
        tt = t_l[b]
        gt = bits > tt
        n1 = jnp.sum(gt.astype(jnp.int32))
        m = _TOPK - n1
        tie = bits == tt
        tie_rank = _excl_prefix_rowmajor(tie.astype(jnp.int32))
        sel = gt | (tie & (tie_rank < m))
        rank = _excl_prefix_rowmajor(sel.astype(jnp.int32))
        rank_ref[b] = jnp.where(sel, rank, jnp.int32(H * W))


def _run_select(attn, beta, interpret=False):
    B, H, W = attn.shape
    return pl.pallas_call(
        _select_kernel,
        out_shape=(
            jax.ShapeDtypeStruct((B, H, W), jnp.int32),
            jax.ShapeDtypeStruct((B, H, W), jnp.float32),
            jax.ShapeDtypeStruct((B, H, W), jnp.float32),
        ),
        in_specs=[
            pl.BlockSpec((B, H, W), lambda: (0, 0, 0)),
            pl.BlockSpec((1, 1), lambda: (0, 0)),
        ],
        out_specs=(
            pl.BlockSpec((B, H, W), lambda: (0, 0, 0)),
            pl.BlockSpec((B, H, W), lambda: (0, 0, 0)),
            pl.BlockSpec((B, H, W), lambda: (0, 0, 0)),
        ),
        interpret=interpret,
    )(attn, beta.reshape(1, 1))


def _make_compact_kernel(B, HW):
    TPI = 8
    CHUNK = HW // TPI
    mesh = plsc.VectorSubcoreMesh(core_axis_name="c", subcore_axis_name="s")

    @functools.partial(
        pl.kernel,
        out_type=[jax.ShapeDtypeStruct((B * TPI * _SLOTS,), jnp.float32)
                  for _ in range(4)],
        mesh=mesh,
        compiler_params=pltpu.CompilerParams(needs_layout_passes=False),
        scratch_types=[
            pltpu.VMEM((CHUNK,), jnp.int32),
            pltpu.VMEM((CHUNK,), jnp.float32),
            pltpu.VMEM((CHUNK,), jnp.float32),
            pltpu.VMEM((_SLOTS,), jnp.float32),
            pltpu.VMEM((_SLOTS,), jnp.float32),
            pltpu.VMEM((_SLOTS,), jnp.float32),
            pltpu.VMEM((_SLOTS,), jnp.float32),
        ],
    )
    def compact(rank_hbm, axm_hbm, aym_hbm, cxo, cyo, axo, ayo,
                rankv, axv, ayv, cxb, cyb, axb, ayb):
        nc = plsc.get_sparse_core_info().num_cores
        wid = lax.axis_index("s") * nc + lax.axis_index("c")
        base = wid * CHUNK

        neg1 = jnp.full((16,), -1.0, jnp.float32)

        def init_body(j, carry):
            cxb[pl.ds(j * 16, 16)] = neg1
            cyb[pl.ds(j * 16, 16)] = neg1
            axb[pl.ds(j * 16, 16)] = neg1
            ayb[pl.ds(j * 16, 16)] = neg1
            return carry

        lax.fori_loop(0, _SLOTS // 16, init_body, 0)

        pltpu.sync_copy(rank_hbm.at[pl.ds(base, CHUNK)], rankv)
        pltpu.sync_copy(axm_hbm.at[pl.ds(base, CHUNK)], axv)
        pltpu.sync_copy(aym_hbm.at[pl.ds(base, CHUNK)], ayv)
        pos0 = base - (wid // TPI) * HW

        def vec_body(i, carry):
            rv = rankv[pl.ds(i * 16, 16)]
            mask = rv < _SLOTS
            idx = jnp.minimum(rv, _SLOTS - 1)
            pos = lax.iota(jnp.int32, 16) + (pos0 + i * 16)
            cxv = (pos & 127).astype(jnp.float32)
            cyv = (pos >> 7).astype(jnp.float32)
            plsc.store_scatter(cxb, [idx], cxv, mask=mask)
            plsc.store_scatter(cyb, [idx], cyv, mask=mask)
            plsc.store_scatter(axb, [idx], axv[pl.ds(i * 16, 16)], mask=mask)
            plsc.store_scatter(ayb, [idx], ayv[pl.ds(i * 16, 16)], mask=mask)
            return carry

        lax.fori_loop(0, CHUNK // 16, vec_body, 0)
        out_off = wid * _SLOTS
        pltpu.sync_copy(cxb, cxo.at[pl.ds(out_off, _SLOTS)])
        pltpu.sync_copy(cyb, cyo.at[pl.ds(out_off, _SLOTS)])
        pltpu.sync_copy(axb, axo.at[pl.ds(out_off, _SLOTS)])
        pltpu.sync_copy(ayb, ayo.at[pl.ds(out_off, _SLOTS)])

    return compact


def _gauss_kernel(cx_ref, cy_ref, ax_ref, ay_ref, g_ref,
                  cxm, cym, axm, aym, glog_s):
    B, TPI, S, _ = cx_ref.shape
    H, W = g_ref.shape[1], g_ref.shape[2]
    lane_f = jax.lax.broadcasted_iota(jnp.int32, (1, W), 1).astype(jnp.float32)
    col_f = jax.lax.broadcasted_iota(jnp.int32, (H, 1), 0).astype(jnp.float32)

    for b in range(B):
        for ref, mref in ((cx_ref, cxm), (cy_ref, cym),
                          (ax_ref, axm), (ay_ref, aym)):
            acc = ref[b, 0]
            for t in range(1, TPI):
                acc = jnp.maximum(acc, ref[b, t])
            mref[b] = acc

    UNROLL = 4
    for b in range(B):
        glog_s[b] = jnp.full((H, W), -jnp.inf, jnp.float32)

    def body(k2, carry):
        for b in range(B):
            contrib = None
            for u in range(UNROLL):
                k = k2 * UNROLL + u
                cxk = cxm[b, pl.ds(k, 1), :]
                cyk = cym[b, pl.ds(k, 1), :]
                axk = axm[b, pl.ds(k, 1), :]
                ayk = aym[b, pl.ds(k, 1), :]
                dx = lane_f - cxk
                dy = col_f - cyk
                term = (-(dy * dy) * ayk) + (-(dx * dx) * axk)
                contrib = term if contrib is None else jnp.maximum(
                    contrib, term)
            glog_s[b] = jnp.maximum(glog_s[b], contrib)
        return carry

    jax.lax.fori_loop(0, _TOPK // UNROLL, body, 0)
    for b in range(B):
        g_ref[b] = jnp.exp(glog_s[b])


def _run_gauss(cx, cy, ax, ay, B, H, W, interpret=False):
    TPI = 8
    spec = pl.BlockSpec((B, TPI, _SLOTS, 1), lambda: (0, 0, 0, 0))
    scratch = [pltpu.VMEM((B, _SLOTS, 1), jnp.float32) for _ in range(4)]
    scratch.append(pltpu.VMEM((B, H, W), jnp.float32))
    return pl.pallas_call(
        _gauss_kernel,
        out_shape=jax.ShapeDtypeStruct((B, H, W), jnp.float32),
        in_specs=[spec] * 4,
        out_specs=pl.BlockSpec((B, H, W), lambda: (0, 0, 0)),
        scratch_shapes=scratch,
        interpret=interpret,
    )(cx.reshape(B, TPI, _SLOTS, 1), cy.reshape(B, TPI, _SLOTS, 1),
      ax.reshape(B, TPI, _SLOTS, 1), ay.reshape(B, TPI, _SLOTS, 1))


def _mod_kernel(agg_ref, g_ref, gamma_ref, out_ref):
    g = g_ref[0][None, None, :, :]
    a = agg_ref[...]
    out_ref[...] = a + gamma_ref[0, 0] * (g * a)


def _run_mod(agg, g, gamma, interpret=False):
    B, C, H, W = agg.shape
    CB = 128
    return pl.pallas_call(
        _mod_kernel,
        grid=(B, C // CB),
        out_shape=jax.ShapeDtypeStruct((B, C, H, W), jnp.float32),
        in_specs=[
            pl.BlockSpec((1, CB, H, W), lambda b, c: (b, c, 0, 0)),
            pl.BlockSpec((1, H, W), lambda b, c: (b, 0, 0)),
            pl.BlockSpec((1, 1), lambda b, c: (0, 0)),
        ],
        out_specs=pl.BlockSpec((1, CB, H, W), lambda b, c: (b, c, 0, 0)),
        interpret=interpret,
    )(agg, g, gamma.reshape(1, 1))


@jax.jit
def kernel(agg_detection_feats, detection_attn_map, beta, gamma):
    B, C, H, W = agg_detection_feats.shape
    beta = jnp.asarray(beta, jnp.float32)
    gamma = jnp.asarray(gamma, jnp.float32)

    rank, axm, aym = _run_select(detection_attn_map, beta)
    compact = _make_compact_kernel(B, H * W)
    cx, cy, ax, ay = compact(
        rank.reshape(B * H * W), axm.reshape(B * H * W),
        aym.reshape(B * H * W))
    g = _run_gauss(cx, cy, ax, ay, B, H, W)
    return _run_mod(agg_detection_feats, g, gamma)


# speedup vs baseline: 1.1168x; 1.0143x over previous
import functools

import jax
import jax.numpy as jnp
from jax import lax
from jax.experimental import pallas as pl
from jax.experimental.pallas import tpu as pltpu
from jax.experimental.pallas import tpu_sc as plsc

_TAU = 0.5
_DELTA = 3.0
_PEAK = 7
_NEIGH = 9
_TOPK = 200
_SLOTS = 256


def _shift_rows(x, d):
    h, w = x.shape
    if d == 0:
        return x
    z = jnp.zeros((abs(d), w), x.dtype)
    if d > 0:
        return jnp.concatenate([x[d:, :], z], axis=0)
    return jnp.concatenate([z, x[:d, :]], axis=0)


def _shift_cols(x, d):
    h, w = x.shape
    if d == 0:
        return x
    z = jnp.zeros((h, abs(d)), x.dtype)
    if d > 0:
        return jnp.concatenate([x[:, d:], z], axis=1)
    return jnp.concatenate([z, x[:, :d]], axis=1)


def _excl_prefix_rowmajor(x):
    h, w = x.shape
    c = x
    s = 1
    while s < w:
        c = c + _shift_cols(c, -s)
        s *= 2
    row_tot = c[:, w - 1:w]
    t = row_tot
    s = 1
    while s < h:
        t = t + _shift_rows(t, -s)
        s *= 2
    return (t - row_tot) + (c - x)


def _select_kernel(attn_ref, beta_ref, rank_ref, ax_ref, ay_ref):
    B, H, W = attn_ref.shape
    pr = _PEAK // 2
    nr = _NEIGH // 2
    beta = jnp.abs(beta_ref[0, 0]) + 1e-6

    m_all = jnp.maximum(attn_ref[...], 0.0)
    mmax = jnp.max(m_all)
    scale = jnp.where(mmax > 0, 1.0 / (mmax + 1e-8), 1.0)

    bits_l = []
    for b in range(B):
        m_th_raw = m_all[b] * scale
        m_th = jnp.where(m_th_raw > _TAU, m_th_raw, 0.0)

        rp = m_th
        for d in range(1, pr + 1):
            rp = jnp.maximum(
                rp, jnp.maximum(_shift_rows(m_th, d), _shift_rows(m_th, -d)))
        pooled = rp
        for d in range(1, pr + 1):
            pooled = jnp.maximum(
                pooled, jnp.maximum(_shift_cols(rp, d), _shift_cols(rp, -d)))

        is_peak = (m_th == pooled) & (m_th > 0)
        sc0 = jnp.where(is_peak, m_th, 0.0)
        bits_l.append(lax.bitcast_convert_type(sc0, jnp.int32))

        cs = m_th
        for d in range(1, nr + 1):
            cs = cs + _shift_rows(m_th, d) + _shift_rows(m_th, -d)
        ssx = jnp.zeros_like(m_th)
        for d in range(-nr, nr + 1):
            if d != 0:
                ssx = ssx + float(d * d) * _shift_cols(cs, d)
        sw = jnp.maximum(jnp.sqrt(jnp.maximum(ssx, 1e-8)), _DELTA)
        ax_ref[b] = 1.0 / (beta * sw * sw)

        rs = m_th
        for d in range(1, nr + 1):
            rs = rs + _shift_cols(m_th, d) + _shift_cols(m_th, -d)
        ssy = jnp.zeros_like(m_th)
        for d in range(-nr, nr + 1):
            if d != 0:
                ssy = ssy + float(d * d) * _shift_rows(rs, d)
        sh = jnp.maximum(jnp.sqrt(jnp.maximum(ssy, 1e-8)), _DELTA)
        ay_ref[b] = 1.0 / (beta * sh * sh)

    t_l = []
    for b in range(B):
        cnt0 = jnp.sum((bits_l[b] >= jnp.int32(0x3F000000)).astype(jnp.int32))
        t_l.append(jnp.where(cnt0 >= _TOPK, jnp.int32(0x3F000000),
                             jnp.int32(0)))
    for bit in range(23, -1, -1):
        for b in range(B):
            tc = t_l[b] | jnp.int32(1 << bit)
            cnt = jnp.sum((bits_l[b] >= tc).astype(jnp.int32))
            t_l[b] = jnp.where(cnt >= _TOPK, tc, t_l[b])

    for b in range(B):
        bits = bits_l[b]
        tt = t_l[b]
        gt = bits > tt
        n1 = jnp.sum(gt.astype(jnp.int32))
        m = _TOPK - n1
        tie = bits == tt
        tie_rank = _excl_prefix_rowmajor(tie.astype(jnp.int32))
        sel = gt | (tie & (tie_rank < m))
        rank = _excl_prefix_rowmajor(sel.astype(jnp.int32))
        rank_ref[b] = jnp.where(sel, rank, jnp.int32(H * W))


def _run_select(attn, beta, interpret=False):
    B, H, W = attn.shape
    return pl.pallas_call(
        _select_kernel,
        out_shape=(
            jax.ShapeDtypeStruct((B, H, W), jnp.int32),
            jax.ShapeDtypeStruct((B, H, W), jnp.float32),
            jax.ShapeDtypeStruct((B, H, W), jnp.float32),
        ),
        in_specs=[
            pl.BlockSpec((B, H, W), lambda: (0, 0, 0)),
            pl.BlockSpec((1, 1), lambda: (0, 0)),
        ],
        out_specs=(
            pl.BlockSpec((B, H, W), lambda: (0, 0, 0)),
            pl.BlockSpec((B, H, W), lambda: (0, 0, 0)),
            pl.BlockSpec((B, H, W), lambda: (0, 0, 0)),
        ),
        interpret=interpret,
    )(attn, beta.reshape(1, 1))


def _make_compact_kernel(B, HW):
    TPI = 8
    CHUNK = HW // TPI
    mesh = plsc.VectorSubcoreMesh(core_axis_name="c", subcore_axis_name="s")

    @functools.partial(
        pl.kernel,
        out_type=[jax.ShapeDtypeStruct((B * TPI * _SLOTS,), jnp.float32)
                  for _ in range(4)],
        mesh=mesh,
        compiler_params=pltpu.CompilerParams(needs_layout_passes=False),
        scratch_types=[
            pltpu.VMEM((CHUNK,), jnp.int32),
            pltpu.VMEM((CHUNK,), jnp.float32),
            pltpu.VMEM((CHUNK,), jnp.float32),
            pltpu.VMEM((_SLOTS,), jnp.float32),
            pltpu.VMEM((_SLOTS,), jnp.float32),
            pltpu.VMEM((_SLOTS,), jnp.float32),
            pltpu.VMEM((_SLOTS,), jnp.float32),
        ],
    )
    def compact(rank_hbm, axm_hbm, aym_hbm, cxo, cyo, axo, ayo,
                rankv, axv, ayv, cxb, cyb, axb, ayb):
        nc = plsc.get_sparse_core_info().num_cores
        wid = lax.axis_index("s") * nc + lax.axis_index("c")
        base = wid * CHUNK

        neg1 = jnp.full((16,), -1.0, jnp.float32)

        def init_body(j, carry):
            cxb[pl.ds(j * 16, 16)] = neg1
            cyb[pl.ds(j * 16, 16)] = neg1
            axb[pl.ds(j * 16, 16)] = neg1
            ayb[pl.ds(j * 16, 16)] = neg1
            return carry

        lax.fori_loop(0, _SLOTS // 16, init_body, 0)

        pltpu.sync_copy(rank_hbm.at[pl.ds(base, CHUNK)], rankv)
        pltpu.sync_copy(axm_hbm.at[pl.ds(base, CHUNK)], axv)
        pltpu.sync_copy(aym_hbm.at[pl.ds(base, CHUNK)], ayv)
        pos0 = base - (wid // TPI) * HW

        def vec_body(i, carry):
            rv = rankv[pl.ds(i * 16, 16)]
            mask = rv < _SLOTS
            idx = jnp.minimum(rv, _SLOTS - 1)
            pos = lax.iota(jnp.int32, 16) + (pos0 + i * 16)
            cxv = (pos & 127).astype(jnp.float32)
            cyv = (pos >> 7).astype(jnp.float32)
            plsc.store_scatter(cxb, [idx], cxv, mask=mask)
            plsc.store_scatter(cyb, [idx], cyv, mask=mask)
            plsc.store_scatter(axb, [idx], axv[pl.ds(i * 16, 16)], mask=mask)
            plsc.store_scatter(ayb, [idx], ayv[pl.ds(i * 16, 16)], mask=mask)
            return carry

        lax.fori_loop(0, CHUNK // 16, vec_body, 0)
        out_off = wid * _SLOTS
        pltpu.sync_copy(cxb, cxo.at[pl.ds(out_off, _SLOTS)])
        pltpu.sync_copy(cyb, cyo.at[pl.ds(out_off, _SLOTS)])
        pltpu.sync_copy(axb, axo.at[pl.ds(out_off, _SLOTS)])
        pltpu.sync_copy(ayb, ayo.at[pl.ds(out_off, _SLOTS)])

    return compact


def _gauss_kernel(cx_ref, cy_ref, ax_ref, ay_ref, g_ref,
                  cxm, cym, axm, aym, glog_s):
    B, TPI, S, _ = cx_ref.shape
    H, W = g_ref.shape[1], g_ref.shape[2]
    lane_f = jax.lax.broadcasted_iota(jnp.int32, (1, W), 1).astype(jnp.float32)
    col_f = jax.lax.broadcasted_iota(jnp.int32, (H, 1), 0).astype(jnp.float32)

    for b in range(B):
        for ref, mref in ((cx_ref, cxm), (cy_ref, cym),
                          (ax_ref, axm), (ay_ref, aym)):
            acc = ref[b, 0]
            for t in range(1, TPI):
                acc = jnp.maximum(acc, ref[b, t])
            mref[b] = acc

    UNROLL = 8
    for b in range(B):
        glog_s[b] = jnp.full((H, W), -jnp.inf, jnp.float32)

    def body(k2, carry):
        for b in range(B):
            contrib = None
            for u in range(UNROLL):
                k = k2 * UNROLL + u
                cxk = cxm[b, pl.ds(k, 1), :]
                cyk = cym[b, pl.ds(k, 1), :]
                axk = axm[b, pl.ds(k, 1), :]
                ayk = aym[b, pl.ds(k, 1), :]
                dx = lane_f - cxk
                dy = col_f - cyk
                term = (-(dy * dy) * ayk) + (-(dx * dx) * axk)
                contrib = term if contrib is None else jnp.maximum(
                    contrib, term)
            glog_s[b] = jnp.maximum(glog_s[b], contrib)
        return carry

    jax.lax.fori_loop(0, _TOPK // UNROLL, body, 0)
    for b in range(B):
        g_ref[b] = jnp.exp(glog_s[b])


def _run_gauss(cx, cy, ax, ay, B, H, W, interpret=False):
    TPI = 8
    spec = pl.BlockSpec((B, TPI, _SLOTS, 1), lambda: (0, 0, 0, 0))
    scratch = [pltpu.VMEM((B, _SLOTS, 1), jnp.float32) for _ in range(4)]
    scratch.append(pltpu.VMEM((B, H, W), jnp.float32))
    return pl.pallas_call(
        _gauss_kernel,
        out_shape=jax.ShapeDtypeStruct((B, H, W), jnp.float32),
        in_specs=[spec] * 4,
        out_specs=pl.BlockSpec((B, H, W), lambda: (0, 0, 0)),
        scratch_shapes=scratch,
        interpret=interpret,
    )(cx.reshape(B, TPI, _SLOTS, 1), cy.reshape(B, TPI, _SLOTS, 1),
      ax.reshape(B, TPI, _SLOTS, 1), ay.reshape(B, TPI, _SLOTS, 1))


def _mod_kernel(agg_ref, g_ref, gamma_ref, out_ref):
    g = g_ref[0][None, None, :, :]
    a = agg_ref[...]
    out_ref[...] = a + gamma_ref[0, 0] * (g * a)


def _run_mod(agg, g, gamma, interpret=False):
    B, C, H, W = agg.shape
    CB = 128
    return pl.pallas_call(
        _mod_kernel,
        grid=(B, C // CB),
        out_shape=jax.ShapeDtypeStruct((B, C, H, W), jnp.float32),
        in_specs=[
            pl.BlockSpec((1, CB, H, W), lambda b, c: (b, c, 0, 0)),
            pl.BlockSpec((1, H, W), lambda b, c: (b, 0, 0)),
            pl.BlockSpec((1, 1), lambda b, c: (0, 0)),
        ],
        out_specs=pl.BlockSpec((1, CB, H, W), lambda b, c: (b, c, 0, 0)),
        interpret=interpret,
    )(agg, g, gamma.reshape(1, 1))


@jax.jit
def kernel(agg_detection_feats, detection_attn_map, beta, gamma):
    B, C, H, W = agg_detection_feats.shape
    beta = jnp.asarray(beta, jnp.float32)
    gamma = jnp.asarray(gamma, jnp.float32)

    rank, axm, aym = _run_select(detection_attn_map, beta)
    compact = _make_compact_kernel(B, H * W)
    cx, cy, ax, ay = compact(
        rank.reshape(B * H * W), axm.reshape(B * H * W),
        aym.reshape(B * H * W))
    g = _run_gauss(cx, cy, ax, ay, B, H, W)
    return _run_mod(agg_detection_feats, g, gamma)
